# baseline (speedup 1.0000x reference)
import jax
import jax.numpy as jnp
from jax.experimental import pallas as pl
from jax.experimental.pallas import tpu as pltpu

ALPHA_EPS = 1e-12
TOTAL = 8 * 10 * 256 * 256
LANES = 256
ROWS = TOTAL // LANES
BLOCK_R = 1024


CHUNK = 32


def _body(pred_ref, tgt_ref, out_ref, acc_ref):
    i = pl.program_id(0)

    @pl.when(i == 0)
    def _init():
        acc_ref[0] = 0.0

    acc = jnp.zeros((CHUNK, LANES), jnp.float32)
    for j in range(BLOCK_R // CHUNK):
        p = pred_ref[j * CHUNK:(j + 1) * CHUNK, :]
        t = tgt_ref[j * CHUNK:(j + 1) * CHUNK, :]
        one_m_p = 1.0 - p
        one_m_t = 1.0 - t
        nw2 = one_m_t * one_m_t
        neg = -jnp.log(one_m_p + ALPHA_EPS) * (p * p) * (nw2 * nw2)
        pos = -jnp.log(p + ALPHA_EPS) * (one_m_p * one_m_p)
        loss = jnp.where(t == 1.0, pos + neg, neg)
        acc = acc + loss
    acc_ref[0] += jnp.sum(acc)

    @pl.when(i == pl.num_programs(0) - 1)
    def _fin():
        out_ref[0] = acc_ref[0] * (1.0 / TOTAL)


def kernel(pred, target):
    p2 = pred.reshape(ROWS, LANES)
    t2 = target.reshape(ROWS, LANES)
    out = pl.pallas_call(
        _body,
        grid=(ROWS // BLOCK_R,),
        in_specs=[
            pl.BlockSpec((BLOCK_R, LANES), lambda i: (i, 0)),
            pl.BlockSpec((BLOCK_R, LANES), lambda i: (i, 0)),
        ],
        out_specs=pl.BlockSpec(memory_space=pltpu.SMEM),
        out_shape=jax.ShapeDtypeStruct((1,), jnp.float32),
        scratch_shapes=[pltpu.SMEM((1,), jnp.float32)],
    )(p2, t2)
    return out[0]


# speedup vs baseline: 1.2939x; 1.2939x over previous
import jax
import jax.numpy as jnp
from jax.experimental import pallas as pl
from jax.experimental.pallas import tpu as pltpu

ALPHA_EPS = 1e-12
TOTAL = 8 * 10 * 256 * 256
LANES = 256
ROWS = TOTAL // LANES
BLOCK_R = 4096


CHUNK = 32


def _body(pred_ref, tgt_ref, out_ref, acc_ref):
    i = pl.program_id(0)

    @pl.when(i == 0)
    def _init():
        acc_ref[0] = 0.0

    acc = jnp.zeros((CHUNK, LANES), jnp.float32)
    for j in range(BLOCK_R // CHUNK):
        p = pred_ref[j * CHUNK:(j + 1) * CHUNK, :]
        t = tgt_ref[j * CHUNK:(j + 1) * CHUNK, :]
        one_m_p = 1.0 - p
        one_m_t = 1.0 - t
        nw2 = one_m_t * one_m_t
        neg = -jnp.log(one_m_p + ALPHA_EPS) * (p * p) * (nw2 * nw2)
        pos = -jnp.log(p + ALPHA_EPS) * (one_m_p * one_m_p)
        loss = jnp.where(t == 1.0, pos + neg, neg)
        acc = acc + loss
    acc_ref[0] += jnp.sum(acc)

    @pl.when(i == pl.num_programs(0) - 1)
    def _fin():
        out_ref[0] = acc_ref[0] * (1.0 / TOTAL)


def kernel(pred, target):
    p2 = pred.reshape(ROWS, LANES)
    t2 = target.reshape(ROWS, LANES)
    out = pl.pallas_call(
        _body,
        grid=(ROWS // BLOCK_R,),
        in_specs=[
            pl.BlockSpec((BLOCK_R, LANES), lambda i: (i, 0)),
            pl.BlockSpec((BLOCK_R, LANES), lambda i: (i, 0)),
        ],
        out_specs=pl.BlockSpec(memory_space=pltpu.SMEM),
        out_shape=jax.ShapeDtypeStruct((1,), jnp.float32),
        scratch_shapes=[pltpu.SMEM((1,), jnp.float32)],
    )(p2, t2)
    return out[0]
